# baseline (speedup 1.0000x reference)
import functools

import jax
import jax.numpy as jnp
from jax import lax
from jax.experimental import pallas as pl
from jax.experimental.pallas import tpu as pltpu
from jax.experimental.pallas import tpu_sc as plsc

EMBED = 64
BATCH = 16384
NUM_CORES = 2
NUM_SUBCORES = 16
NUM_WORKERS = NUM_CORES * NUM_SUBCORES
ROWS_PER_WORKER = BATCH // NUM_WORKERS
IDX_CHUNKS = 4
IDX_CHUNK = ROWS_PER_WORKER // IDX_CHUNKS
GROUPS = ROWS_PER_WORKER // 16

_mesh = plsc.VectorSubcoreMesh(core_axis_name="c", subcore_axis_name="s")


@functools.partial(
    pl.kernel,
    mesh=_mesh,
    compiler_params=pltpu.CompilerParams(
        needs_layout_passes=False, use_tc_tiling_on_sc=False),
    out_type=jax.ShapeDtypeStruct((BATCH,), jnp.float32),
    scratch_types=[
        pltpu.VMEM((IDX_CHUNKS, IDX_CHUNK), jnp.int32),
        pltpu.VMEM((IDX_CHUNKS, IDX_CHUNK), jnp.int32),
        pltpu.VMEM((ROWS_PER_WORKER, EMBED), jnp.float32),
        pltpu.VMEM((ROWS_PER_WORKER, EMBED), jnp.float32),
        pltpu.VMEM((ROWS_PER_WORKER,), jnp.float32),
        pltpu.SemaphoreType.DMA,
    ],
)
def _sc_dot_kernel(nodes_u_hbm, nodes_v_hbm, user_hbm, item_hbm, out_hbm,
                   idx_u, idx_v, urows, vrows, obuf, sem):
    wid = lax.axis_index("s") * NUM_CORES + lax.axis_index("c")

    pltpu.sync_copy(nodes_u_hbm.at[wid], idx_u)
    pltpu.sync_copy(nodes_v_hbm.at[wid], idx_v)

    copies = []
    for j in range(IDX_CHUNKS):
        dst = pl.ds(j * IDX_CHUNK, IDX_CHUNK)
        copies.append(pltpu.async_copy(user_hbm.at[idx_u.at[j]], urows.at[dst], sem))
        copies.append(pltpu.async_copy(item_hbm.at[idx_v.at[j]], vrows.at[dst], sem))
    for c in copies:
        c.wait()

    lanes = lax.iota(jnp.int32, 16)

    def group_body(g, carry):
        base = pl.multiple_of(g * 16, 16)
        rows = base + lanes
        acc = jnp.zeros((16,), jnp.float32)
        for d in range(EMBED):
            col = jnp.full((16,), d, jnp.int32)
            uc = plsc.load_gather(urows, [rows, col])
            vc = plsc.load_gather(vrows, [rows, col])
            acc = acc + uc * vc
        obuf[pl.ds(base, 16)] = acc
        return carry

    lax.fori_loop(0, GROUPS, group_body, 0)

    pltpu.sync_copy(obuf, out_hbm.at[pl.ds(wid * ROWS_PER_WORKER, ROWS_PER_WORKER)])


def kernel(nodes_u, nodes_v, user_emb, item_emb):
    nu = nodes_u.astype(jnp.int32).reshape(NUM_WORKERS, IDX_CHUNKS, IDX_CHUNK)
    nv = nodes_v.astype(jnp.int32).reshape(NUM_WORKERS, IDX_CHUNKS, IDX_CHUNK)
    out = _sc_dot_kernel(nu, nv, user_emb, item_emb)
    return out.reshape(BATCH, 1, 1)


# speedup vs baseline: 1.5607x; 1.5607x over previous
import functools

import jax
import jax.numpy as jnp
from jax import lax
from jax.experimental import pallas as pl
from jax.experimental.pallas import tpu as pltpu
from jax.experimental.pallas import tpu_sc as plsc

EMBED = 64
BATCH = 16384
NUM_CORES = 2
NUM_SUBCORES = 16
NUM_WORKERS = NUM_CORES * NUM_SUBCORES
ROWS_PER_WORKER = BATCH // NUM_WORKERS
PASS_ROWS = ROWS_PER_WORKER // 2
PASS_GROUPS = PASS_ROWS // 16

_mesh = plsc.VectorSubcoreMesh(core_axis_name="c", subcore_axis_name="s")


@functools.partial(
    pl.kernel,
    mesh=_mesh,
    compiler_params=pltpu.CompilerParams(needs_layout_passes=False),
    out_type=jax.ShapeDtypeStruct((BATCH,), jnp.float32),
    scratch_types=[
        pltpu.VMEM((ROWS_PER_WORKER,), jnp.int32),
        pltpu.VMEM((ROWS_PER_WORKER,), jnp.int32),
        pltpu.VMEM((PASS_ROWS, EMBED), jnp.float32),
        pltpu.VMEM((PASS_ROWS, EMBED), jnp.float32),
        pltpu.VMEM((ROWS_PER_WORKER,), jnp.float32),
        pltpu.SemaphoreType.DMA,
    ],
)
def _sc_dot_kernel(nodes_u_hbm, nodes_v_hbm, user_hbm, item_hbm, out_hbm,
                   idx_u, idx_v, urows, vrows, obuf, sem):
    wid = lax.axis_index("s") * NUM_CORES + lax.axis_index("c")
    base_row = pl.multiple_of(wid * ROWS_PER_WORKER, ROWS_PER_WORKER)

    pltpu.sync_copy(nodes_u_hbm.at[pl.ds(base_row, ROWS_PER_WORKER)], idx_u)
    pltpu.sync_copy(nodes_v_hbm.at[pl.ds(base_row, ROWS_PER_WORKER)], idx_v)

    lanes = lax.iota(jnp.int32, 16)

    for p in range(2):
        poff = p * PASS_ROWS

        def issue_body(k, carry):
            kbase = pl.multiple_of(k * 16, 16)
            iu = idx_u[pl.ds(poff + kbase, 16)]
            iv = idx_v[pl.ds(poff + kbase, 16)]
            for j in range(16):
                dst = pl.ds(kbase + j, 1)
                pltpu.async_copy(user_hbm.at[pl.ds(iu[j], 1), :],
                                 urows.at[dst, :], sem)
                pltpu.async_copy(item_hbm.at[pl.ds(iv[j], 1), :],
                                 vrows.at[dst, :], sem)
            return carry

        lax.fori_loop(0, PASS_GROUPS, issue_body, 0)

        def drain_body(k, carry):
            kbase = pl.multiple_of(k * 16, 16)
            for j in range(16):
                dst = pl.ds(kbase + j, 1)
                pltpu.make_async_copy(user_hbm.at[pl.ds(0, 1), :],
                                      urows.at[dst, :], sem).wait()
                pltpu.make_async_copy(item_hbm.at[pl.ds(0, 1), :],
                                      vrows.at[dst, :], sem).wait()
            return carry

        lax.fori_loop(0, PASS_GROUPS, drain_body, 0)

        def group_body(g, carry):
            gbase = pl.multiple_of(g * 16, 16)
            rows = gbase + lanes
            acc = jnp.zeros((16,), jnp.float32)
            for d in range(EMBED):
                col = jnp.full((16,), d, jnp.int32)
                acc = acc + (plsc.load_gather(urows, [rows, col])
                             * plsc.load_gather(vrows, [rows, col]))
            obuf[pl.ds(poff + gbase, 16)] = acc
            return carry

        lax.fori_loop(0, PASS_GROUPS, group_body, 0)

    pltpu.sync_copy(obuf, out_hbm.at[pl.ds(base_row, ROWS_PER_WORKER)])


def kernel(nodes_u, nodes_v, user_emb, item_emb):
    nu = nodes_u.astype(jnp.int32)
    nv = nodes_v.astype(jnp.int32)
    out = _sc_dot_kernel(nu, nv, user_emb, item_emb)
    return out.reshape(BATCH, 1, 1)
